# Initial kernel scaffold; baseline (speedup 1.0000x reference)
import jax
import jax.numpy as jnp
from jax.experimental import pallas as pl


def kernel(nodes, coords, edge_index, hn_W1, hn_b1, hn_W2, hn_b2, hc_W1, hc_b1, hc_W2, hc_b2, mp_W1, mp_b1, mp_W2, mp_b2):
    raise NotImplementedError("write your pallas kernel here")


# speedup vs baseline: 1.1511x; 1.1511x over previous
import functools

import jax
import jax.numpy as jnp
from jax import lax
from jax.experimental import pallas as pl
from jax.experimental.pallas import tpu as pltpu
from jax.experimental.pallas import tpu_sc as plsc

N = 10000
E = 160000
D = 256
NT_N = 10
TN = N // NT_N
NT_E = 160
TE = E // NT_E

_NEG_INF = float("-inf")


def _enc_body(nodes_ref, coords_ref, w1n, b1n, w2n, b2n, w1c, b1c, w2c, b2c,
              h_ref, hc_ref):
    t = jnp.maximum(
        jnp.dot(nodes_ref[...], w1n[...], preferred_element_type=jnp.float32)
        + b1n[...], 0.0)
    h_ref[...] = jnp.dot(t, w2n[...], preferred_element_type=jnp.float32) + b2n[...]
    t2 = jnp.maximum(
        jnp.dot(coords_ref[...], w1c[...], preferred_element_type=jnp.float32)
        + b1c[...], 0.0)
    hc_ref[...] = jnp.dot(t2, w2c[...], preferred_element_type=jnp.float32) + b2c[...]


def _encode(nodes, coords8, hn_W1, hn_b1, hn_W2, hn_b2, hc_W1p, hc_b1, hc_W2, hc_b2):
    full = lambda shape: pl.BlockSpec(shape, lambda i: (0, 0))
    return pl.pallas_call(
        _enc_body,
        grid=(NT_N,),
        in_specs=[
            pl.BlockSpec((TN, 128), lambda i: (i, 0)),
            pl.BlockSpec((TN, 8), lambda i: (i, 0)),
            full((128, D)), full((1, D)), full((D, D)), full((1, D)),
            full((8, D)), full((1, D)), full((D, D)), full((1, D)),
        ],
        out_specs=[
            pl.BlockSpec((TN, D), lambda i: (i, 0)),
            pl.BlockSpec((TN, D), lambda i: (i, 0)),
        ],
        out_shape=[
            jax.ShapeDtypeStruct((N, D), jnp.float32),
            jax.ShapeDtypeStruct((N, D), jnp.float32),
        ],
    )(nodes, coords8, hn_W1, hn_b1.reshape(1, D), hn_W2, hn_b2.reshape(1, D),
      hc_W1p, hc_b1.reshape(1, D), hc_W2, hc_b2.reshape(1, D))


def _st_body(h_ref, hc_ref, wa, wb, wc, b1, s_ref, t_ref):
    h = h_ref[...]
    hcwc = jnp.dot(hc_ref[...], wc[...], preferred_element_type=jnp.float32)
    s_ref[...] = (jnp.dot(h, wa[...], preferred_element_type=jnp.float32)
                  + hcwc + b1[...])
    t_ref[...] = (jnp.dot(h, wb[...], preferred_element_type=jnp.float32)
                  - hcwc)


def _node_transform(h, hc, wa, wb, wc, b1):
    full = lambda: pl.BlockSpec((D, D), lambda i: (0, 0))
    return pl.pallas_call(
        _st_body,
        grid=(NT_N,),
        in_specs=[
            pl.BlockSpec((TN, D), lambda i: (i, 0)),
            pl.BlockSpec((TN, D), lambda i: (i, 0)),
            full(), full(), full(),
            pl.BlockSpec((1, D), lambda i: (0, 0)),
        ],
        out_specs=[
            pl.BlockSpec((TN, D), lambda i: (i, 0)),
            pl.BlockSpec((TN, D), lambda i: (i, 0)),
        ],
        out_shape=[
            jax.ShapeDtypeStruct((N, D), jnp.float32),
            jax.ShapeDtypeStruct((N, D), jnp.float32),
        ],
    )(h, hc, wa, wb, wc, b1.reshape(1, D))


_CH = 200


def _gather2(S, T, src, dst):
    info = plsc.get_sparse_core_info()
    nc, ns = info.num_cores, info.num_subcores
    nw = nc * ns
    epw = E // nw
    nch = epw // _CH
    mesh = plsc.VectorSubcoreMesh(core_axis_name="c", subcore_axis_name="s")

    @functools.partial(
        pl.kernel,
        out_type=(jax.ShapeDtypeStruct((E, D), jnp.float32),
                  jax.ShapeDtypeStruct((E, D), jnp.float32)),
        mesh=mesh,
        scratch_types=[
            pltpu.VMEM((_CH,), jnp.int32),
            pltpu.VMEM((_CH,), jnp.int32),
            pltpu.VMEM((_CH, D), jnp.float32),
            pltpu.VMEM((_CH, D), jnp.float32),
            pltpu.SemaphoreType.DMA,
            pltpu.SemaphoreType.DMA,
        ],
    )
    def k(S_hbm, T_hbm, src_hbm, dst_hbm, P_hbm, Q_hbm,
          si_v, di_v, sr_v, dr_v, sem1, sem2):
        wid = lax.axis_index("s") * nc + lax.axis_index("c")
        base_w = wid * epw
        for c in range(nch):
            base = base_w + c * _CH
            pltpu.sync_copy(src_hbm.at[pl.ds(base, _CH)], si_v)
            pltpu.sync_copy(dst_hbm.at[pl.ds(base, _CH)], di_v)
            cp1 = pltpu.async_copy(S_hbm.at[si_v], sr_v, sem1)
            cp2 = pltpu.async_copy(T_hbm.at[di_v], dr_v, sem2)
            cp1.wait()
            cp2.wait()
            pltpu.sync_copy(sr_v, P_hbm.at[pl.ds(base, _CH)])
            pltpu.sync_copy(dr_v, Q_hbm.at[pl.ds(base, _CH)])

    return k(S, T, src, dst)


def _edge_body(p_ref, q_ref, w2, b2, m_ref):
    a = jnp.maximum(p_ref[...] + q_ref[...], 0.0)
    m_ref[...] = jnp.dot(a, w2[...], preferred_element_type=jnp.float32) + b2[...]


def _edge_mlp(P, Q, w2, b2):
    return pl.pallas_call(
        _edge_body,
        grid=(NT_E,),
        in_specs=[
            pl.BlockSpec((TE, D), lambda i: (i, 0)),
            pl.BlockSpec((TE, D), lambda i: (i, 0)),
            pl.BlockSpec((D, D), lambda i: (0, 0)),
            pl.BlockSpec((1, D), lambda i: (0, 0)),
        ],
        out_specs=pl.BlockSpec((TE, D), lambda i: (i, 0)),
        out_shape=jax.ShapeDtypeStruct((E, D), jnp.float32),
    )(P, Q, w2, b2.reshape(1, D))


def _scat_body(dst_ref, m_ref, h_ref, out_ref, acc_ref):
    @pl.when(pl.program_id(0) == 0)
    def _init():
        acc_ref[...] = jnp.full((N, D), _NEG_INF, jnp.float32)

    def upd(j, _):
        d = dst_ref[0, 0, j]
        row = m_ref[pl.ds(j, 1), :]
        acc_ref[pl.ds(d, 1), :] = jnp.maximum(acc_ref[pl.ds(d, 1), :], row)
        return 0

    lax.fori_loop(0, TE, upd, 0)

    @pl.when(pl.program_id(0) == NT_E - 1)
    def _fin():
        agg = acc_ref[...]
        agg = jnp.where(jnp.isneginf(agg), 0.0, agg)
        out_ref[...] = h_ref[...] + agg


def _scatter_max_update(dst2, M, h):
    return pl.pallas_call(
        _scat_body,
        grid=(NT_E,),
        in_specs=[
            pl.BlockSpec((1, 1, TE), lambda i: (i, 0, 0), memory_space=pltpu.SMEM),
            pl.BlockSpec((TE, D), lambda i: (i, 0)),
            pl.BlockSpec((N, D), lambda i: (0, 0)),
        ],
        out_specs=pl.BlockSpec((N, D), lambda i: (0, 0)),
        out_shape=jax.ShapeDtypeStruct((N, D), jnp.float32),
        scratch_shapes=[pltpu.VMEM((N, D), jnp.float32)],
    )(dst2, M, h)


def kernel(nodes, coords, edge_index, hn_W1, hn_b1, hn_W2, hn_b2,
           hc_W1, hc_b1, hc_W2, hc_b2, mp_W1, mp_b1, mp_W2, mp_b2):
    src = edge_index[0]
    dst = edge_index[1]
    dst2 = dst.reshape(NT_E, 1, TE)
    coords8 = jnp.pad(coords, ((0, 0), (0, 5)))
    hc_W1p = jnp.pad(hc_W1, ((0, 5), (0, 0)))

    h, hcv = _encode(nodes, coords8, hn_W1, hn_b1, hn_W2, hn_b2,
                     hc_W1p, hc_b1, hc_W2, hc_b2)

    for i in range(3):
        wa = mp_W1[i, 0:D, :]
        wb = mp_W1[i, D:2 * D, :]
        wc = mp_W1[i, 2 * D:3 * D, :]
        S, T = _node_transform(h, hcv, wa, wb, wc, mp_b1[i])
        P, Q = _gather2(S, T, src, dst)
        M = _edge_mlp(P, Q, mp_W2[i], mp_b2[i])
        h = _scatter_max_update(dst2, M, h)
    return h
